# baseline (speedup 1.0000x reference)
import functools

import jax
import jax.numpy as jnp
import numpy as np
from jax import lax
from jax.experimental import pallas as pl
from jax.experimental.pallas import tpu as pltpu
from jax.experimental.pallas import tpu_sc as plsc

N = 10000
E = 320000
HID = 128
H = 8
C = 16
B = 16
ACCW = 144

W = 80
NTILES = 32
EPT = E // NTILES
NCH = EPT // W
RPT = N // 16

ROWB = 1000
NROWB = N // ROWB


def _edge_body(qa_hbm, kv_hbm, src_hbm, dst_hbm, ea_hbm, wep_hbm, zeros_hbm,
               zerosd_hbm, outm_hbm, outd_hbm, srci, dsti, qabuf, kvbuf, eabuf,
               msgbuf, denbuf, wepbuf, accm, accd):
    c = lax.axis_index("c")
    s = lax.axis_index("s")
    wid = s * 2 + c

    pltpu.sync_copy(zeros_hbm, accm.at[pl.ds(s * RPT, RPT), :])
    pltpu.sync_copy(zerosd_hbm, accd.at[pl.ds(s * RPT, RPT), :])
    pltpu.sync_copy(wep_hbm, wepbuf)
    plsc.subcore_barrier()

    lane = lax.iota(jnp.int32, 16)
    zeros16i = jnp.zeros((16,), jnp.int32)
    ones16i = jnp.ones((16,), jnp.int32)
    lane_lt8 = lane < 8
    p8 = (lane ^ 8)[:, None]
    spl = [jnp.full((16, 1), h, jnp.int32) for h in range(H)]
    gdn = lax.GatherDimensionNumbers(offset_dims=(), collapsed_slice_dims=(0,),
                                     start_index_map=(0,))
    shuf = lambda t, p: lax.gather(
        t, p, gdn, (1,), mode=lax.GatherScatterMode.PROMISE_IN_BOUNDS)

    @pl.loop(0, NCH)
    def _chunk(ci):
        base = wid * EPT + ci * W
        pltpu.sync_copy(src_hbm.at[pl.ds(base, W)], srci)
        pltpu.sync_copy(dst_hbm.at[pl.ds(base, W)], dsti)
        pltpu.sync_copy(ea_hbm.at[pl.ds(base, W), :], eabuf)
        pltpu.sync_copy(kv_hbm.at[srci], kvbuf)
        pltpu.sync_copy(qa_hbm.at[dsti], qabuf)

        we0 = [wepbuf[0, pl.ds(h * 16, 16)] for h in range(H)]
        we1 = [wepbuf[1, pl.ds(h * 16, 16)] for h in range(H)]
        be = [wepbuf[2, pl.ds(h * 16, 16)] for h in range(H)]

        @pl.loop(0, W)
        def _edge(w):
            wv = jnp.full((16,), w, jnp.int32)
            ea0 = plsc.load_gather(eabuf, [wv, zeros16i])
            ea1 = plsc.load_gather(eabuf, [wv, ones16i])
            dots = [qabuf[w, pl.ds(j * 16, 16)] * kvbuf[w, pl.ds(j * 16, 16)]
                    for j in range(8)]
            acc = ((dots[0] + dots[1]) + (dots[2] + dots[3])) + \
                  ((dots[4] + dots[5]) + (dots[6] + dots[7]))
            acc = acc + shuf(acc, p8)
            qx01 = qabuf[w, pl.ds(128, 16)]
            qxb = qabuf[w, pl.ds(144, 16)]
            alpha = acc + ea0 * qx01 + ea1 * shuf(qx01, p8) + qxb
            expa = jnp.exp(jnp.minimum(alpha, 60.0))
            plsc.store_scatter(denbuf, [wv, lane], expa, mask=lane_lt8)
            for h in range(H):
                e = ea0 * we0[h] + ea1 * we1[h] + be[h]
                eh = shuf(expa, spl[h])
                msgbuf[w, pl.ds(h * 16, 16)] = \
                    (kvbuf[w, pl.ds(128 + h * 16, 16)] + e) * eh

        pltpu.sync_copy(msgbuf, accm.at[dsti], add=True)
        pltpu.sync_copy(denbuf, accd.at[dsti], add=True)

    plsc.subcore_barrier()
    pltpu.sync_copy(accm.at[pl.ds(s * RPT, RPT), :],
                    outm_hbm.at[c, pl.ds(s * RPT, RPT), :])
    pltpu.sync_copy(accd.at[pl.ds(s * RPT, RPT), :],
                    outd_hbm.at[c, pl.ds(s * RPT, RPT), :])


def _edge_pass(qa, kv, src, dst, ea, wepack, zeros, zerosd):
    mesh = plsc.VectorSubcoreMesh(core_axis_name="c", subcore_axis_name="s")
    k = functools.partial(
        pl.kernel,
        mesh=mesh,
        compiler_params=pltpu.CompilerParams(use_tc_tiling_on_sc=False,
                                             needs_layout_passes=False),
        out_type=[
            jax.ShapeDtypeStruct((2, N, HID), jnp.float32),
            jax.ShapeDtypeStruct((2, N, H), jnp.float32),
        ],
        scratch_types=[
            pltpu.VMEM((W,), jnp.int32),
            pltpu.VMEM((W,), jnp.int32),
            pltpu.VMEM((W, 160), jnp.float32),
            pltpu.VMEM((W, 2 * HID), jnp.float32),
            pltpu.VMEM((W, 2), jnp.float32),
            pltpu.VMEM((W, HID), jnp.float32),
            pltpu.VMEM((W, H), jnp.float32),
            pltpu.VMEM((3, HID), jnp.float32),
            pltpu.VMEM_SHARED((N, HID), jnp.float32),
            pltpu.VMEM_SHARED((N, H), jnp.float32),
        ],
    )(_edge_body)
    return k(qa, kv, src, dst, ea, wepack, zeros, zerosd)


def _ln(x, g, b):
    m = jnp.mean(x, axis=-1, keepdims=True)
    v = jnp.mean((x - m) ** 2, axis=-1, keepdims=True)
    return (x - m) / jnp.sqrt(v + 1e-5) * g + b


def _embed_body(x_ref, pe_ref, win_ref, bin_ref, g_ref, b_ref, wpe_ref,
                bpe_ref, o_ref):
    h = _ln(x_ref[...] @ win_ref[...] + bin_ref[...], g_ref[...], b_ref[...])
    o_ref[...] = h + pe_ref[...] @ wpe_ref[...] + bpe_ref[...]


def _embed(x, pe, win, bin_, g, b, wpe, bpe):
    full = lambda shp: pl.BlockSpec(shp, lambda i: (0,) * len(shp))
    return pl.pallas_call(
        _embed_body,
        grid=(NROWB,),
        in_specs=[
            pl.BlockSpec((ROWB, HID), lambda i: (i, 0)),
            pl.BlockSpec((ROWB, 8), lambda i: (i, 0)),
            full((HID, HID)), full((1, HID)), full((1, HID)), full((1, HID)),
            full((8, HID)), full((1, HID)),
        ],
        out_specs=pl.BlockSpec((ROWB, HID), lambda i: (i, 0)),
        out_shape=jax.ShapeDtypeStruct((N, HID), jnp.float32),
    )(x, pe, win, bin_, g, b, wpe, bpe)


def _proj_body(h_ref, wa_ref, ba_ref, wb_ref, bb_ref, qa_ref, kv_ref):
    hh = h_ref[...]
    qa_ref[...] = hh @ wa_ref[...] + ba_ref[...]
    kv_ref[...] = hh @ wb_ref[...] + bb_ref[...]


def _proj(h, wa, ba, wb, bb):
    full = lambda shp: pl.BlockSpec(shp, lambda i: (0,) * len(shp))
    return pl.pallas_call(
        _proj_body,
        grid=(NROWB,),
        in_specs=[
            pl.BlockSpec((ROWB, HID), lambda i: (i, 0)),
            full((HID, 160)), full((1, 160)),
            full((HID, 2 * HID)), full((1, 2 * HID)),
        ],
        out_specs=[
            pl.BlockSpec((ROWB, 160), lambda i: (i, 0)),
            pl.BlockSpec((ROWB, 2 * HID), lambda i: (i, 0)),
        ],
        out_shape=[
            jax.ShapeDtypeStruct((N, 160), jnp.float32),
            jax.ShapeDtypeStruct((N, 2 * HID), jnp.float32),
        ],
    )(h, wa, ba, wb, bb)


def _post_body(accm_ref, accd_ref, h_ref, e8_ref, ws_ref, bs_ref, g_ref,
               b_ref, w1_ref, b1_ref, w2_ref, b2_ref, o_ref):
    a = accm_ref[0] + accm_ref[1]
    d = accd_ref[0] + accd_ref[1]
    denx = d @ e8_ref[...]
    attn = a / (denx + 1e-16)
    h = h_ref[...]
    h2 = attn + h @ ws_ref[...] + bs_ref[...]
    hh = _ln(h + h2, g_ref[...], b_ref[...])
    f = jnp.maximum(hh @ w1_ref[...] + b1_ref[...], 0.0) @ w2_ref[...] \
        + b2_ref[...]
    o_ref[...] = _ln(hh + f, g_ref[...], b_ref[...])


def _post(accm, accd, h, e8, ws, bs, g, b, w1, b1, w2, b2):
    full = lambda shp: pl.BlockSpec(shp, lambda i: (0,) * len(shp))
    return pl.pallas_call(
        _post_body,
        grid=(NROWB,),
        in_specs=[
            pl.BlockSpec((2, ROWB, HID), lambda i: (0, i, 0)),
            pl.BlockSpec((2, ROWB, H), lambda i: (0, i, 0)),
            pl.BlockSpec((ROWB, HID), lambda i: (i, 0)),
            full((H, HID)), full((HID, HID)), full((1, HID)),
            full((1, HID)), full((1, HID)),
            full((HID, 4 * HID)), full((1, 4 * HID)),
            full((4 * HID, HID)), full((1, HID)),
        ],
        out_specs=pl.BlockSpec((ROWB, HID), lambda i: (i, 0)),
        out_shape=jax.ShapeDtypeStruct((N, HID), jnp.float32),
    )(accm, accd, h, e8, ws, bs, g, b, w1, b1, w2, b2)


def _pool_body(h_ref, bt_ref, wt1_ref, bt1_ref, wt2_ref, bt2_ref, wg1_ref,
               bg1_ref, wg2_ref, bg2_ref, t_ref, g_ref, pacc, cacc):
    i = pl.program_id(0)

    @pl.when(i == 0)
    def _():
        pacc[...] = jnp.zeros((B, HID), jnp.float32)
        cacc[...] = jnp.zeros((B, HID), jnp.float32)

    bb = jnp.broadcast_to(bt_ref[0], (B, ROWB))
    ids = lax.broadcasted_iota(jnp.int32, (B, ROWB), 0).astype(jnp.float32)
    oh = (ids == bb).astype(jnp.float32)
    pacc[...] += oh @ h_ref[...]
    cacc[...] += jnp.broadcast_to(jnp.sum(oh, axis=1, keepdims=True), (B, HID))

    @pl.when(i == NROWB - 1)
    def _():
        pooled = pacc[...] / jnp.maximum(cacc[...], 1.0)
        z = jnp.maximum(pooled @ wt1_ref[...] + bt1_ref[...], 0.0)
        tv = z @ wt2_ref[...] + bt2_ref[...]
        t_ref[...] = jnp.broadcast_to(tv[:, :1], (B, HID))
        zg = jnp.maximum(pooled @ wg1_ref[...] + bg1_ref[...], 0.0)
        gv = jax.nn.sigmoid(zg @ wg2_ref[...] + bg2_ref[...]) * 2.0
        g_ref[...] = jnp.broadcast_to(gv[:, :1], (B, HID))


def _pool(h, batchf, wt1, bt1, wt2, bt2, wg1, bg1, wg2, bg2):
    full = lambda shp: pl.BlockSpec(shp, lambda i: (0,) * len(shp))
    out = pl.pallas_call(
        _pool_body,
        grid=(NROWB,),
        in_specs=[
            pl.BlockSpec((ROWB, HID), lambda i: (i, 0)),
            pl.BlockSpec((1, 1, ROWB), lambda i: (i, 0, 0)),
            full((HID, 64)), full((1, 64)), full((64, HID)), full((1, HID)),
            full((HID, 64)), full((1, 64)), full((64, HID)), full((1, HID)),
        ],
        out_specs=[
            pl.BlockSpec((B, HID), lambda i: (0, 0)),
            pl.BlockSpec((B, HID), lambda i: (0, 0)),
        ],
        out_shape=[
            jax.ShapeDtypeStruct((B, HID), jnp.float32),
            jax.ShapeDtypeStruct((B, HID), jnp.float32),
        ],
        scratch_shapes=[
            pltpu.VMEM((B, HID), jnp.float32),
            pltpu.VMEM((B, HID), jnp.float32),
        ],
    )(h, batchf, wt1, bt1, wt2, bt2, wg1, bg1, wg2, bg2)
    return out


_E8 = np.kron(np.eye(H, dtype=np.float32), np.ones((1, C), np.float32))

_PI = np.empty((HID,), np.int64)
for _j in range(8):
    for _p in range(2):
        for _h in range(H):
            _PI[_j * 16 + _p * 8 + _h] = _h * 16 + 2 * _j + _p
_HSEL = np.kron(np.eye(H, dtype=np.float32), np.ones((C, 1), np.float32))


def kernel(x, edge_index, edge_attr, batch, pe, params):
    with jax.default_matmul_precision("highest"):
        return _kernel_impl(x, edge_index, edge_attr, batch, pe, params)


def _kernel_impl(x, edge_index, edge_attr, batch, pe, params):
    p = params
    src = edge_index[0].astype(jnp.int32)
    dst = edge_index[1].astype(jnp.int32)
    ea = edge_attr.astype(jnp.float32)
    e8 = jnp.asarray(_E8)
    zeros = jnp.zeros((RPT, HID), jnp.float32)
    zerosd = jnp.zeros((RPT, H), jnp.float32)
    r = lambda v: v.reshape(1, -1)

    h = _embed(x, pe, p['Win'], r(p['bin']), r(p['g_in']), r(p['b_in']),
               p['Wpe'], r(p['bpe']))

    hsel = jnp.asarray(_HSEL)
    for lp in p['layers']:
        wepack = jnp.concatenate([lp['We'], lp['be'].reshape(1, -1)], axis=0)
        m0 = lp['We'][0][:, None] * hsel
        m1 = lp['We'][1][:, None] * hsel
        m2 = lp['be'][:, None] * hsel
        mfull = jnp.concatenate([m0, m1, m2, jnp.zeros((HID, 8))], axis=1)
        wa = jnp.concatenate([lp['Wq'][:, _PI], lp['Wq'] @ mfull],
                             axis=1) * 0.25
        ba = jnp.concatenate([lp['bq'][_PI], lp['bq'] @ mfull]) * 0.25
        wb = jnp.concatenate([lp['Wk'][:, _PI], lp['Wv']], axis=1)
        bb = jnp.concatenate([lp['bk'][_PI], lp['bv']])
        qa, kv = _proj(h, wa, ba.reshape(1, -1), wb, bb.reshape(1, -1))
        accm, accd = _edge_pass(qa, kv, src, dst, ea, wepack, zeros, zerosd)
        h = _post(accm, accd, h, e8, lp['Ws'], r(lp['bs']), r(lp['ln_g']),
                  r(lp['ln_b']), lp['W1'], r(lp['b1']), lp['W2'], r(lp['b2']))

    batchf = batch.astype(jnp.float32).reshape(NROWB, 1, ROWB)
    pad2 = lambda w: jnp.pad(w, ((0, 0), (0, HID - w.shape[1])))
    padb = lambda v: jnp.pad(v.reshape(1, -1), ((0, 0), (0, HID - v.shape[0])))
    t2, g2 = _pool(h, batchf, p['Wt1'], r(p['bt1']), pad2(p['Wt2']),
                   padb(p['bt2']), p['Wg1'], r(p['bg1']), pad2(p['Wg2']),
                   padb(p['bg2']))
    return (t2[:, 0], g2[:, 0])


# speedup vs baseline: 1.0575x; 1.0575x over previous
import functools

import jax
import jax.numpy as jnp
import numpy as np
from jax import lax
from jax.experimental import pallas as pl
from jax.experimental.pallas import tpu as pltpu
from jax.experimental.pallas import tpu_sc as plsc

N = 10000
E = 320000
HID = 128
H = 8
C = 16
B = 16
ACCW = 144

W = 80
NTILES = 32
EPT = E // NTILES
NCH = EPT // W
RPT = N // 16

ROWB = 1000
NROWB = N // ROWB


def _edge_body(qa_hbm, kv_hbm, src_hbm, dst_hbm, ea_hbm, wep_hbm, zeros_hbm,
               zerosd_hbm, outm_hbm, outd_hbm, srci, dsti, qabuf, kvbuf, eabuf,
               msgbuf, denbuf, wepbuf, accm, accd):
    c = lax.axis_index("c")
    s = lax.axis_index("s")
    wid = s * 2 + c

    pltpu.sync_copy(zeros_hbm, accm.at[pl.ds(s * RPT, RPT), :])
    pltpu.sync_copy(zerosd_hbm, accd.at[pl.ds(s * RPT, RPT), :])
    pltpu.sync_copy(wep_hbm, wepbuf)
    plsc.subcore_barrier()

    lane = lax.iota(jnp.int32, 16)
    zeros16i = jnp.zeros((16,), jnp.int32)
    ones16i = jnp.ones((16,), jnp.int32)
    lane_lt8 = lane < 8
    lane7 = lane & 7
    p8 = (lane ^ 8)[:, None]
    spl = [jnp.full((16, 1), h, jnp.int32) for h in range(H)]
    gdn = lax.GatherDimensionNumbers(offset_dims=(), collapsed_slice_dims=(0,),
                                     start_index_map=(0,))
    shuf = lambda t, p: lax.gather(
        t, p, gdn, (1,), mode=lax.GatherScatterMode.PROMISE_IN_BOUNDS)

    @pl.loop(0, NCH)
    def _chunk(ci):
        base = wid * EPT + ci * W
        pltpu.sync_copy(src_hbm.at[pl.ds(base, W)], srci)
        pltpu.sync_copy(dst_hbm.at[pl.ds(base, W)], dsti)
        pltpu.sync_copy(ea_hbm.at[pl.ds(base, W), :], eabuf)
        pltpu.sync_copy(kv_hbm.at[srci], kvbuf)
        pltpu.sync_copy(qa_hbm.at[dsti], qabuf)

        we0 = [wepbuf[0, pl.ds(h * 16, 16)] for h in range(H)]
        we1 = [wepbuf[1, pl.ds(h * 16, 16)] for h in range(H)]
        be = [wepbuf[2, pl.ds(h * 16, 16)] for h in range(H)]

        @pl.loop(0, W, step=2)
        def _edge(w0):
            ws = [w0, w0 + 1]
            wvs = [jnp.full((16,), w, jnp.int32) for w in ws]
            ea0s = [plsc.load_gather(eabuf, [wv, zeros16i]) for wv in wvs]
            ea1s = [plsc.load_gather(eabuf, [wv, ones16i]) for wv in wvs]
            dots = [[qabuf[w, pl.ds(j * 16, 16)] * kvbuf[w, pl.ds(j * 16, 16)]
                     for j in range(8)] for w in ws]
            accs = [((d[0] + d[1]) + (d[2] + d[3])) +
                    ((d[4] + d[5]) + (d[6] + d[7])) for d in dots]
            accs = [a + shuf(a, p8) for a in accs]
            qx01s = [qabuf[w, pl.ds(128, 16)] for w in ws]
            qxbs = [qabuf[w, pl.ds(144, 16)] for w in ws]
            alphas = [a + e0 * qx + e1 * shuf(qx, p8) + qb
                      for a, e0, e1, qx, qb
                      in zip(accs, ea0s, ea1s, qx01s, qxbs)]
            expas = [jnp.exp(jnp.minimum(a, 60.0)) for a in alphas]
            rowi = jnp.where(lane_lt8, wvs[0], wvs[1])
            comb = jnp.where(lane_lt8, expas[0], shuf(expas[1], p8))
            plsc.store_scatter(denbuf, [rowi, lane7], comb)
            for h in range(H):
                for i in range(2):
                    e = ea0s[i] * we0[h] + ea1s[i] * we1[h] + be[h]
                    msgbuf[ws[i], pl.ds(h * 16, 16)] = \
                        (kvbuf[ws[i], pl.ds(128 + h * 16, 16)] + e) * \
                        shuf(expas[i], spl[h])

        pltpu.sync_copy(msgbuf, accm.at[dsti], add=True)
        pltpu.sync_copy(denbuf, accd.at[dsti], add=True)

    plsc.subcore_barrier()
    pltpu.sync_copy(accm.at[pl.ds(s * RPT, RPT), :],
                    outm_hbm.at[c, pl.ds(s * RPT, RPT), :])
    pltpu.sync_copy(accd.at[pl.ds(s * RPT, RPT), :],
                    outd_hbm.at[c, pl.ds(s * RPT, RPT), :])


def _edge_pass(qa, kv, src, dst, ea, wepack, zeros, zerosd):
    mesh = plsc.VectorSubcoreMesh(core_axis_name="c", subcore_axis_name="s")
    k = functools.partial(
        pl.kernel,
        mesh=mesh,
        compiler_params=pltpu.CompilerParams(use_tc_tiling_on_sc=False,
                                             needs_layout_passes=False),
        out_type=[
            jax.ShapeDtypeStruct((2, N, HID), jnp.float32),
            jax.ShapeDtypeStruct((2, N, H), jnp.float32),
        ],
        scratch_types=[
            pltpu.VMEM((W,), jnp.int32),
            pltpu.VMEM((W,), jnp.int32),
            pltpu.VMEM((W, 160), jnp.float32),
            pltpu.VMEM((W, 2 * HID), jnp.float32),
            pltpu.VMEM((W, 2), jnp.float32),
            pltpu.VMEM((W, HID), jnp.float32),
            pltpu.VMEM((W, H), jnp.float32),
            pltpu.VMEM((3, HID), jnp.float32),
            pltpu.VMEM_SHARED((N, HID), jnp.float32),
            pltpu.VMEM_SHARED((N, H), jnp.float32),
        ],
    )(_edge_body)
    return k(qa, kv, src, dst, ea, wepack, zeros, zerosd)


def _ln(x, g, b):
    m = jnp.mean(x, axis=-1, keepdims=True)
    v = jnp.mean((x - m) ** 2, axis=-1, keepdims=True)
    return (x - m) / jnp.sqrt(v + 1e-5) * g + b


def _embed_body(x_ref, pe_ref, win_ref, bin_ref, g_ref, b_ref, wpe_ref,
                bpe_ref, o_ref):
    h = _ln(x_ref[...] @ win_ref[...] + bin_ref[...], g_ref[...], b_ref[...])
    o_ref[...] = h + pe_ref[...] @ wpe_ref[...] + bpe_ref[...]


def _embed(x, pe, win, bin_, g, b, wpe, bpe):
    full = lambda shp: pl.BlockSpec(shp, lambda i: (0,) * len(shp))
    return pl.pallas_call(
        _embed_body,
        grid=(NROWB,),
        in_specs=[
            pl.BlockSpec((ROWB, HID), lambda i: (i, 0)),
            pl.BlockSpec((ROWB, 8), lambda i: (i, 0)),
            full((HID, HID)), full((1, HID)), full((1, HID)), full((1, HID)),
            full((8, HID)), full((1, HID)),
        ],
        out_specs=pl.BlockSpec((ROWB, HID), lambda i: (i, 0)),
        out_shape=jax.ShapeDtypeStruct((N, HID), jnp.float32),
    )(x, pe, win, bin_, g, b, wpe, bpe)


def _proj_body(h_ref, wa_ref, ba_ref, wb_ref, bb_ref, qa_ref, kv_ref):
    hh = h_ref[...]
    qa_ref[...] = hh @ wa_ref[...] + ba_ref[...]
    kv_ref[...] = hh @ wb_ref[...] + bb_ref[...]


def _proj(h, wa, ba, wb, bb):
    full = lambda shp: pl.BlockSpec(shp, lambda i: (0,) * len(shp))
    return pl.pallas_call(
        _proj_body,
        grid=(NROWB,),
        in_specs=[
            pl.BlockSpec((ROWB, HID), lambda i: (i, 0)),
            full((HID, 160)), full((1, 160)),
            full((HID, 2 * HID)), full((1, 2 * HID)),
        ],
        out_specs=[
            pl.BlockSpec((ROWB, 160), lambda i: (i, 0)),
            pl.BlockSpec((ROWB, 2 * HID), lambda i: (i, 0)),
        ],
        out_shape=[
            jax.ShapeDtypeStruct((N, 160), jnp.float32),
            jax.ShapeDtypeStruct((N, 2 * HID), jnp.float32),
        ],
    )(h, wa, ba, wb, bb)


def _post_body(accm_ref, accd_ref, h_ref, e8_ref, ws_ref, bs_ref, g_ref,
               b_ref, w1_ref, b1_ref, w2_ref, b2_ref, o_ref):
    a = accm_ref[0] + accm_ref[1]
    d = accd_ref[0] + accd_ref[1]
    denx = d @ e8_ref[...]
    attn = a / (denx + 1e-16)
    h = h_ref[...]
    h2 = attn + h @ ws_ref[...] + bs_ref[...]
    hh = _ln(h + h2, g_ref[...], b_ref[...])
    f = jnp.maximum(hh @ w1_ref[...] + b1_ref[...], 0.0) @ w2_ref[...] \
        + b2_ref[...]
    o_ref[...] = _ln(hh + f, g_ref[...], b_ref[...])


def _post(accm, accd, h, e8, ws, bs, g, b, w1, b1, w2, b2):
    full = lambda shp: pl.BlockSpec(shp, lambda i: (0,) * len(shp))
    return pl.pallas_call(
        _post_body,
        grid=(NROWB,),
        in_specs=[
            pl.BlockSpec((2, ROWB, HID), lambda i: (0, i, 0)),
            pl.BlockSpec((2, ROWB, H), lambda i: (0, i, 0)),
            pl.BlockSpec((ROWB, HID), lambda i: (i, 0)),
            full((H, HID)), full((HID, HID)), full((1, HID)),
            full((1, HID)), full((1, HID)),
            full((HID, 4 * HID)), full((1, 4 * HID)),
            full((4 * HID, HID)), full((1, HID)),
        ],
        out_specs=pl.BlockSpec((ROWB, HID), lambda i: (i, 0)),
        out_shape=jax.ShapeDtypeStruct((N, HID), jnp.float32),
    )(accm, accd, h, e8, ws, bs, g, b, w1, b1, w2, b2)


def _pool_body(h_ref, bt_ref, wt1_ref, bt1_ref, wt2_ref, bt2_ref, wg1_ref,
               bg1_ref, wg2_ref, bg2_ref, t_ref, g_ref, pacc, cacc):
    i = pl.program_id(0)

    @pl.when(i == 0)
    def _():
        pacc[...] = jnp.zeros((B, HID), jnp.float32)
        cacc[...] = jnp.zeros((B, HID), jnp.float32)

    bb = jnp.broadcast_to(bt_ref[0], (B, ROWB))
    ids = lax.broadcasted_iota(jnp.int32, (B, ROWB), 0).astype(jnp.float32)
    oh = (ids == bb).astype(jnp.float32)
    pacc[...] += oh @ h_ref[...]
    cacc[...] += jnp.broadcast_to(jnp.sum(oh, axis=1, keepdims=True), (B, HID))

    @pl.when(i == NROWB - 1)
    def _():
        pooled = pacc[...] / jnp.maximum(cacc[...], 1.0)
        z = jnp.maximum(pooled @ wt1_ref[...] + bt1_ref[...], 0.0)
        tv = z @ wt2_ref[...] + bt2_ref[...]
        t_ref[...] = jnp.broadcast_to(tv[:, :1], (B, HID))
        zg = jnp.maximum(pooled @ wg1_ref[...] + bg1_ref[...], 0.0)
        gv = jax.nn.sigmoid(zg @ wg2_ref[...] + bg2_ref[...]) * 2.0
        g_ref[...] = jnp.broadcast_to(gv[:, :1], (B, HID))


def _pool(h, batchf, wt1, bt1, wt2, bt2, wg1, bg1, wg2, bg2):
    full = lambda shp: pl.BlockSpec(shp, lambda i: (0,) * len(shp))
    out = pl.pallas_call(
        _pool_body,
        grid=(NROWB,),
        in_specs=[
            pl.BlockSpec((ROWB, HID), lambda i: (i, 0)),
            pl.BlockSpec((1, 1, ROWB), lambda i: (i, 0, 0)),
            full((HID, 64)), full((1, 64)), full((64, HID)), full((1, HID)),
            full((HID, 64)), full((1, 64)), full((64, HID)), full((1, HID)),
        ],
        out_specs=[
            pl.BlockSpec((B, HID), lambda i: (0, 0)),
            pl.BlockSpec((B, HID), lambda i: (0, 0)),
        ],
        out_shape=[
            jax.ShapeDtypeStruct((B, HID), jnp.float32),
            jax.ShapeDtypeStruct((B, HID), jnp.float32),
        ],
        scratch_shapes=[
            pltpu.VMEM((B, HID), jnp.float32),
            pltpu.VMEM((B, HID), jnp.float32),
        ],
    )(h, batchf, wt1, bt1, wt2, bt2, wg1, bg1, wg2, bg2)
    return out


_E8 = np.kron(np.eye(H, dtype=np.float32), np.ones((1, C), np.float32))

_PI = np.empty((HID,), np.int64)
for _j in range(8):
    for _p in range(2):
        for _h in range(H):
            _PI[_j * 16 + _p * 8 + _h] = _h * 16 + 2 * _j + _p
_HSEL = np.kron(np.eye(H, dtype=np.float32), np.ones((C, 1), np.float32))


def kernel(x, edge_index, edge_attr, batch, pe, params):
    with jax.default_matmul_precision("highest"):
        return _kernel_impl(x, edge_index, edge_attr, batch, pe, params)


def _kernel_impl(x, edge_index, edge_attr, batch, pe, params):
    p = params
    src = edge_index[0].astype(jnp.int32)
    dst = edge_index[1].astype(jnp.int32)
    ea = edge_attr.astype(jnp.float32)
    e8 = jnp.asarray(_E8)
    zeros = jnp.zeros((RPT, HID), jnp.float32)
    zerosd = jnp.zeros((RPT, H), jnp.float32)
    r = lambda v: v.reshape(1, -1)

    h = _embed(x, pe, p['Win'], r(p['bin']), r(p['g_in']), r(p['b_in']),
               p['Wpe'], r(p['bpe']))

    hsel = jnp.asarray(_HSEL)
    for lp in p['layers']:
        wepack = jnp.concatenate([lp['We'], lp['be'].reshape(1, -1)], axis=0)
        m0 = lp['We'][0][:, None] * hsel
        m1 = lp['We'][1][:, None] * hsel
        m2 = lp['be'][:, None] * hsel
        mfull = jnp.concatenate([m0, m1, m2, jnp.zeros((HID, 8))], axis=1)
        wa = jnp.concatenate([lp['Wq'][:, _PI], lp['Wq'] @ mfull],
                             axis=1) * 0.25
        ba = jnp.concatenate([lp['bq'][_PI], lp['bq'] @ mfull]) * 0.25
        wb = jnp.concatenate([lp['Wk'][:, _PI], lp['Wv']], axis=1)
        bb = jnp.concatenate([lp['bk'][_PI], lp['bv']])
        qa, kv = _proj(h, wa, ba.reshape(1, -1), wb, bb.reshape(1, -1))
        accm, accd = _edge_pass(qa, kv, src, dst, ea, wepack, zeros, zerosd)
        h = _post(accm, accd, h, e8, lp['Ws'], r(lp['bs']), r(lp['ln_g']),
                  r(lp['ln_b']), lp['W1'], r(lp['b1']), lp['W2'], r(lp['b2']))

    batchf = batch.astype(jnp.float32).reshape(NROWB, 1, ROWB)
    pad2 = lambda w: jnp.pad(w, ((0, 0), (0, HID - w.shape[1])))
    padb = lambda v: jnp.pad(v.reshape(1, -1), ((0, 0), (0, HID - v.shape[0])))
    t2, g2 = _pool(h, batchf, p['Wt1'], r(p['bt1']), pad2(p['Wt2']),
                   padb(p['bt2']), p['Wg1'], r(p['bg1']), pad2(p['Wg2']),
                   padb(p['bg2']))
    return (t2[:, 0], g2[:, 0])
